# baseline (speedup 1.0000x reference)
import functools

import jax
import jax.numpy as jnp
from jax import lax
from jax.experimental import pallas as pl
from jax.experimental.pallas import tpu as pltpu
from jax.experimental.pallas import tpu_sc as plsc

NC = 2
NS = 16
SB = 128
SPLIT_NUM, SPLIT_DEN = 10, 10


def _make_sc_segsum(n_pad, e_pad, d, with_deg):
    kb = e_pad // (SB * NC * NS)
    kb0 = (2 * kb * SPLIT_NUM // (16 * SPLIT_DEN)) * 16
    kb1 = 2 * kb - kb0
    rows_per_tile = n_pad // NS
    mesh = plsc.VectorSubcoreMesh(core_axis_name="c", subcore_axis_name="s")

    out_type = [jax.ShapeDtypeStruct((NC * n_pad, d), jnp.float32)]
    if with_deg:
        out_type.append(jax.ShapeDtypeStruct((NC * n_pad, d), jnp.float32))

    cb = 16
    scratch = [
        pltpu.VMEM((cb, SB), jnp.int32),
        pltpu.VMEM((cb, SB), jnp.int32),
        pltpu.VMEM((2, SB, d), jnp.float32),
        pltpu.VMEM_SHARED((n_pad, d), jnp.float32),
        pltpu.SemaphoreType.DMA,
        pltpu.SemaphoreType.DMA,
    ]
    def body(y_hbm, src_hbm, dst_hbm, zero_hbm, ones_hbm, out_hbm, *rest):
        if with_deg:
            deg_hbm = rest[0]
            src_v, dst_v, rows_v, acc, sem0, sem1 = rest[1:]
        else:
            src_v, dst_v, rows_v, acc, sem0, sem1 = rest
        sems = (sem0, sem1)
        c = lax.axis_index("c")
        s = lax.axis_index("s")
        wid = c * NS + s
        base = s * rows_per_tile
        out_base = c * n_pad + base

        def zero_my_slice():
            off = 0
            while off < rows_per_tile:
                sz = min(SB, rows_per_tile - off)
                pltpu.sync_copy(zero_hbm.at[pl.ds(off, sz)],
                                acc.at[pl.ds(base + off, sz)])
                off += sz

        def copy_out_my_slice(dest):
            off = 0
            while off < rows_per_tile:
                sz = min(SB, rows_per_tile - off)
                pltpu.sync_copy(acc.at[pl.ds(base + off, sz)],
                                dest.at[pl.ds(out_base + off, sz)])
                off += sz

        zero_my_slice()
        plsc.subcore_barrier()

        tile_blk = jnp.where(c == 0, s * kb0, NS * kb0 + s * kb1)
        nch = jnp.where(c == 0, kb0 // cb, kb1 // cb)

        def step(cc, carry):
            blk = tile_blk + cc * cb
            pltpu.sync_copy(src_hbm.at[pl.ds(blk, cb)], src_v)
            pltpu.sync_copy(dst_hbm.at[pl.ds(blk, cb)], dst_v)
            pending = pltpu.async_copy(y_hbm.at[src_v.at[0]], rows_v.at[0],
                                       sems[0])
            for j in range(cb):
                p = j % 2
                if j + 1 < cb:
                    nxt = pltpu.async_copy(y_hbm.at[src_v.at[j + 1]],
                                           rows_v.at[(j + 1) % 2],
                                           sems[(j + 1) % 2])
                else:
                    nxt = None
                pending.wait()
                pltpu.sync_copy(rows_v.at[p], acc.at[dst_v.at[j]], add=True)
                pending = nxt
            return carry

        lax.fori_loop(0, nch, step, 0)
        plsc.subcore_barrier()
        copy_out_my_slice(out_hbm)

        if with_deg:
            plsc.subcore_barrier()
            ones_v = rows_v.at[0]
            pltpu.sync_copy(ones_hbm, ones_v)
            zero_my_slice()
            plsc.subcore_barrier()

            def dstep(cc, carry):
                blk = wid * kb + cc * cb
                pltpu.sync_copy(dst_hbm.at[pl.ds(blk, cb)], dst_v)
                for j in range(cb):
                    pltpu.sync_copy(ones_v, acc.at[dst_v.at[j]], add=True)
                return carry

            lax.fori_loop(0, kb // cb, dstep, 0)
            plsc.subcore_barrier()
            copy_out_my_slice(deg_hbm)

    return pl.kernel(body, out_type=tuple(out_type), mesh=mesh,
                     scratch_types=scratch)


def _layer_body(with_w2, m_a, m_b, h_in, d_a, d_b, W, b, g, be, W2, out_ref):
    m = m_a[...] + m_b[...]
    d = d_a[...] + d_b[...]
    h = (m + h_in[...]) / (d + 1.0)
    if with_w2:
        h = lax.dot_general(h, W[...], (((1,), (1,)), ((), ())),
                            preferred_element_type=jnp.float32)
    t = h + b[...]
    mu = jnp.mean(t, axis=-1, keepdims=True)
    var = jnp.mean((t - mu) ** 2, axis=-1, keepdims=True)
    t = (t - mu) * lax.rsqrt(var + 1e-5) * g[...] + be[...]
    t = jnp.where(t > 0, t, jnp.exp(jnp.minimum(t, 0.0)) - 1.0)
    if with_w2:
        t = lax.dot_general(t, W2[...], (((1,), (1,)), ((), ())),
                            preferred_element_type=jnp.float32)
    out_ref[...] = t


def _make_tc_layer(n, d, with_w2, block_rows=1000):
    grid = n // block_rows
    row_spec = pl.BlockSpec((block_rows, d), lambda i: (i, 0))
    deg_spec = pl.BlockSpec((block_rows, 1), lambda i: (i, 0))
    full_spec = pl.BlockSpec((d, d), lambda i: (0, 0))
    vec_spec = pl.BlockSpec((1, d), lambda i: (0, 0))
    return pl.pallas_call(
        functools.partial(_layer_body, with_w2),
        grid=grid,
        in_specs=[row_spec, row_spec, row_spec, deg_spec, deg_spec,
                  full_spec, vec_spec, vec_spec, vec_spec, full_spec],
        out_specs=row_spec,
        out_shape=jax.ShapeDtypeStruct((n, d), jnp.float32),
    )


def kernel(features, edge_index, W1, b1, g1, be1, W2, b2, g2, be2):
    n, d = features.shape
    e = edge_index.shape[1]

    chunk = SB * NC * NS * 8
    e_pad = ((e + chunk - 1) // chunk) * chunk
    n_pad = ((n + 1 + NS * 8 - 1) // (NS * 8)) * (NS * 8)

    src = edge_index[0]
    dst = edge_index[1]
    pad = e_pad - e
    src2d = jnp.concatenate([src, jnp.zeros((pad,), jnp.int32)]).reshape(-1, SB)
    dst2d = jnp.concatenate([dst, jnp.full((pad,), n, jnp.int32)]).reshape(-1, SB)
    zeros = jnp.zeros((n_pad // NS, d), jnp.float32)
    ones = jnp.ones((SB, d), jnp.float32)

    segsum_deg = _make_sc_segsum(n_pad, e_pad, d, with_deg=True)
    segsum = _make_sc_segsum(n_pad, e_pad, d, with_deg=False)
    layer1 = _make_tc_layer(n, d, with_w2=True)
    layer2 = _make_tc_layer(n, d, with_w2=False)

    m1, degp = segsum_deg(features, src2d, dst2d, zeros, ones)
    d_a = degp[0:n, 0:1]
    d_b = degp[n_pad:n_pad + n, 0:1]
    W1r = W1.reshape(d, d)
    y2 = layer1(m1[0:n], m1[n_pad:n_pad + n], features, d_a, d_b,
                W1r, b1.reshape(1, d), g1.reshape(1, d), be1.reshape(1, d), W2)

    (m2,) = segsum(y2, src2d, dst2d, zeros, ones)
    out = layer2(m2[0:n], m2[n_pad:n_pad + n], y2, d_a, d_b,
                 W2, b2.reshape(1, d), g2.reshape(1, d), be2.reshape(1, d), W2)
    return out


# speedup vs baseline: 1.3831x; 1.3831x over previous
import functools

import jax
import jax.numpy as jnp
from jax import lax
from jax.experimental import pallas as pl
from jax.experimental.pallas import tpu as pltpu
from jax.experimental.pallas import tpu_sc as plsc

NC = 2
NS = 16
SB = 128
SPLIT_NUM, SPLIT_DEN = 9, 10


def _make_sc_segsum(n_pad, e_pad, d, with_deg):
    kb = e_pad // (SB * NC * NS)
    kb0 = (2 * kb * SPLIT_NUM // (16 * SPLIT_DEN)) * 16
    kb1 = 2 * kb - kb0
    rows_per_tile = n_pad // NS
    mesh = plsc.VectorSubcoreMesh(core_axis_name="c", subcore_axis_name="s")

    out_type = [jax.ShapeDtypeStruct((NC * n_pad, d), jnp.float32)]
    if with_deg:
        out_type.append(jax.ShapeDtypeStruct((NC * n_pad, d), jnp.float32))

    cb = 16
    scratch = [
        pltpu.VMEM((cb, SB), jnp.int32),
        pltpu.VMEM((cb, SB), jnp.int32),
        pltpu.VMEM((2, SB, d), jnp.float32),
        pltpu.VMEM_SHARED((n_pad, d), jnp.float32),
        pltpu.SemaphoreType.DMA,
        pltpu.SemaphoreType.DMA,
    ]
    def body(y_hbm, src_hbm, dst_hbm, zero_hbm, ones_hbm, out_hbm, *rest):
        if with_deg:
            deg_hbm = rest[0]
            src_v, dst_v, rows_v, acc, sem0, sem1 = rest[1:]
        else:
            src_v, dst_v, rows_v, acc, sem0, sem1 = rest
        sems = (sem0, sem1)
        c = lax.axis_index("c")
        s = lax.axis_index("s")
        wid = c * NS + s
        base = s * rows_per_tile
        out_base = c * n_pad + base

        def zero_my_slice():
            descs = []
            off = 0
            while off < rows_per_tile:
                sz = min(SB, rows_per_tile - off)
                descs.append(pltpu.async_copy(
                    zero_hbm.at[pl.ds(off, sz)],
                    acc.at[pl.ds(base + off, sz)], sem0))
                off += sz
            for dd in descs:
                dd.wait()

        def copy_out_my_slice(dest):
            descs = []
            off = 0
            while off < rows_per_tile:
                sz = min(SB, rows_per_tile - off)
                descs.append(pltpu.async_copy(
                    acc.at[pl.ds(base + off, sz)],
                    dest.at[pl.ds(out_base + off, sz)], sem0))
                off += sz
            for dd in descs:
                dd.wait()

        zero_my_slice()
        plsc.subcore_barrier()

        tile_blk = jnp.where(c == 0, s * kb0, NS * kb0 + s * kb1)
        nch = jnp.where(c == 0, kb0 // cb, kb1 // cb)

        def step(cc, carry):
            blk = tile_blk + cc * cb
            pltpu.sync_copy(src_hbm.at[pl.ds(blk, cb)], src_v)
            pltpu.sync_copy(dst_hbm.at[pl.ds(blk, cb)], dst_v)
            pending = pltpu.async_copy(y_hbm.at[src_v.at[0]], rows_v.at[0],
                                       sems[0])
            for j in range(cb):
                p = j % 2
                if j + 1 < cb:
                    nxt = pltpu.async_copy(y_hbm.at[src_v.at[j + 1]],
                                           rows_v.at[(j + 1) % 2],
                                           sems[(j + 1) % 2])
                else:
                    nxt = None
                pending.wait()
                pltpu.sync_copy(rows_v.at[p], acc.at[dst_v.at[j]], add=True)
                pending = nxt
            return carry

        lax.fori_loop(0, nch, step, 0)
        plsc.subcore_barrier()
        copy_out_my_slice(out_hbm)

        if with_deg:
            plsc.subcore_barrier()
            ones_v = rows_v.at[0]
            pltpu.sync_copy(ones_hbm, ones_v)
            zero_my_slice()
            plsc.subcore_barrier()

            def dstep(cc, carry):
                blk = wid * kb + cc * cb
                pltpu.sync_copy(dst_hbm.at[pl.ds(blk, cb)], dst_v)
                for j in range(cb):
                    pltpu.sync_copy(ones_v, acc.at[dst_v.at[j]], add=True)
                return carry

            lax.fori_loop(0, kb // cb, dstep, 0)
            plsc.subcore_barrier()
            copy_out_my_slice(deg_hbm)

    return pl.kernel(body, out_type=tuple(out_type), mesh=mesh,
                     scratch_types=scratch)


def _layer_body(with_w2, m_a, m_b, h_in, d_a, d_b, W, b, g, be, W2, out_ref):
    m = m_a[...] + m_b[...]
    d = d_a[...] + d_b[...]
    h = (m + h_in[...]) / (d + 1.0)
    if with_w2:
        h = lax.dot_general(h, W[...], (((1,), (1,)), ((), ())),
                            preferred_element_type=jnp.float32)
    t = h + b[...]
    mu = jnp.mean(t, axis=-1, keepdims=True)
    var = jnp.mean((t - mu) ** 2, axis=-1, keepdims=True)
    t = (t - mu) * lax.rsqrt(var + 1e-5) * g[...] + be[...]
    t = jnp.where(t > 0, t, jnp.exp(jnp.minimum(t, 0.0)) - 1.0)
    if with_w2:
        t = lax.dot_general(t, W2[...], (((1,), (1,)), ((), ())),
                            preferred_element_type=jnp.float32)
    out_ref[...] = t


def _make_tc_layer(n, d, with_w2, block_rows=1000):
    grid = n // block_rows
    row_spec = pl.BlockSpec((block_rows, d), lambda i: (i, 0))
    deg_spec = pl.BlockSpec((block_rows, 1), lambda i: (i, 0))
    full_spec = pl.BlockSpec((d, d), lambda i: (0, 0))
    vec_spec = pl.BlockSpec((1, d), lambda i: (0, 0))
    return pl.pallas_call(
        functools.partial(_layer_body, with_w2),
        grid=grid,
        in_specs=[row_spec, row_spec, row_spec, deg_spec, deg_spec,
                  full_spec, vec_spec, vec_spec, vec_spec, full_spec],
        out_specs=row_spec,
        out_shape=jax.ShapeDtypeStruct((n, d), jnp.float32),
    )


def kernel(features, edge_index, W1, b1, g1, be1, W2, b2, g2, be2):
    n, d = features.shape
    e = edge_index.shape[1]

    chunk = SB * NC * NS * 8
    e_pad = ((e + chunk - 1) // chunk) * chunk
    n_pad = ((n + 1 + NS * 8 - 1) // (NS * 8)) * (NS * 8)

    src = edge_index[0]
    dst = edge_index[1]
    pad = e_pad - e
    src2d = jnp.concatenate([src, jnp.zeros((pad,), jnp.int32)]).reshape(-1, SB)
    dst2d = jnp.concatenate([dst, jnp.full((pad,), n, jnp.int32)]).reshape(-1, SB)
    zeros = jnp.zeros((n_pad // NS, d), jnp.float32)
    ones = jnp.ones((SB, d), jnp.float32)

    segsum_deg = _make_sc_segsum(n_pad, e_pad, d, with_deg=True)
    segsum = _make_sc_segsum(n_pad, e_pad, d, with_deg=False)
    layer1 = _make_tc_layer(n, d, with_w2=True)
    layer2 = _make_tc_layer(n, d, with_w2=False)

    m1, degp = segsum_deg(features, src2d, dst2d, zeros, ones)
    d_a = degp[0:n, 0:1]
    d_b = degp[n_pad:n_pad + n, 0:1]
    W1r = W1.reshape(d, d)
    y2 = layer1(m1[0:n], m1[n_pad:n_pad + n], features, d_a, d_b,
                W1r, b1.reshape(1, d), g1.reshape(1, d), be1.reshape(1, d), W2)

    (m2,) = segsum(y2, src2d, dst2d, zeros, ones)
    out = layer2(m2[0:n], m2[n_pad:n_pad + n], y2, d_a, d_b,
                 W2, b2.reshape(1, d), g2.reshape(1, d), be2.reshape(1, d), W2)
    return out


# speedup vs baseline: 1.3999x; 1.0122x over previous
import functools

import jax
import jax.numpy as jnp
from jax import lax
from jax.experimental import pallas as pl
from jax.experimental.pallas import tpu as pltpu
from jax.experimental.pallas import tpu_sc as plsc

NC = 2
NS = 16
SB = 128
SPLIT_NUM, SPLIT_DEN = 19, 20


def _make_sc_segsum(n_pad, e_pad, d, with_deg):
    kb = e_pad // (SB * NC * NS)
    kb0 = (2 * kb * SPLIT_NUM // (8 * SPLIT_DEN)) * 8
    kb1 = 2 * kb - kb0
    rows_per_tile = n_pad // NS
    mesh = plsc.VectorSubcoreMesh(core_axis_name="c", subcore_axis_name="s")

    out_type = [jax.ShapeDtypeStruct((NC * n_pad, d), jnp.float32)]
    if with_deg:
        out_type.append(jax.ShapeDtypeStruct((NC * n_pad, d), jnp.float32))

    cb = 8
    scratch = [
        pltpu.VMEM((cb, SB), jnp.int32),
        pltpu.VMEM((cb, SB), jnp.int32),
        pltpu.VMEM((2, SB, d), jnp.float32),
        pltpu.VMEM_SHARED((n_pad, d), jnp.float32),
        pltpu.SemaphoreType.DMA,
        pltpu.SemaphoreType.DMA,
    ]
    def body(y_hbm, src_hbm, dst_hbm, zero_hbm, ones_hbm, out_hbm, *rest):
        if with_deg:
            deg_hbm = rest[0]
            src_v, dst_v, rows_v, acc, sem0, sem1 = rest[1:]
        else:
            src_v, dst_v, rows_v, acc, sem0, sem1 = rest
        sems = (sem0, sem1)
        c = lax.axis_index("c")
        s = lax.axis_index("s")
        wid = c * NS + s
        base = s * rows_per_tile
        out_base = c * n_pad + base

        def zero_my_slice():
            descs = []
            off = 0
            while off < rows_per_tile:
                sz = min(SB, rows_per_tile - off)
                descs.append(pltpu.async_copy(
                    zero_hbm.at[pl.ds(off, sz)],
                    acc.at[pl.ds(base + off, sz)], sem0))
                off += sz
            for dd in descs:
                dd.wait()

        def copy_out_my_slice(dest):
            descs = []
            off = 0
            while off < rows_per_tile:
                sz = min(SB, rows_per_tile - off)
                descs.append(pltpu.async_copy(
                    acc.at[pl.ds(base + off, sz)],
                    dest.at[pl.ds(out_base + off, sz)], sem0))
                off += sz
            for dd in descs:
                dd.wait()

        zero_my_slice()
        plsc.subcore_barrier()

        tile_blk = jnp.where(c == 0, s * kb0, NS * kb0 + s * kb1)
        nch = jnp.where(c == 0, kb0 // cb, kb1 // cb)

        def step(cc, carry):
            blk = tile_blk + cc * cb
            pltpu.sync_copy(src_hbm.at[pl.ds(blk, cb)], src_v)
            pltpu.sync_copy(dst_hbm.at[pl.ds(blk, cb)], dst_v)
            pending = pltpu.async_copy(y_hbm.at[src_v.at[0]], rows_v.at[0],
                                       sems[0])
            for j in range(cb):
                p = j % 2
                if j + 1 < cb:
                    nxt = pltpu.async_copy(y_hbm.at[src_v.at[j + 1]],
                                           rows_v.at[(j + 1) % 2],
                                           sems[(j + 1) % 2])
                else:
                    nxt = None
                pending.wait()
                pltpu.sync_copy(rows_v.at[p], acc.at[dst_v.at[j]], add=True)
                pending = nxt
            return carry

        lax.fori_loop(0, nch, step, 0)
        plsc.subcore_barrier()
        copy_out_my_slice(out_hbm)

        if with_deg:
            plsc.subcore_barrier()
            ones_v = rows_v.at[0]
            pltpu.sync_copy(ones_hbm, ones_v)
            zero_my_slice()
            plsc.subcore_barrier()

            def dstep(cc, carry):
                blk = wid * kb + cc * cb
                pltpu.sync_copy(dst_hbm.at[pl.ds(blk, cb)], dst_v)
                for j in range(cb):
                    pltpu.sync_copy(ones_v, acc.at[dst_v.at[j]], add=True)
                return carry

            lax.fori_loop(0, kb // cb, dstep, 0)
            plsc.subcore_barrier()
            copy_out_my_slice(deg_hbm)

    return pl.kernel(body, out_type=tuple(out_type), mesh=mesh,
                     scratch_types=scratch)


def _layer_body(with_w2, m_a, m_b, h_in, d_a, d_b, W, b, g, be, W2, out_ref):
    m = m_a[...] + m_b[...]
    d = d_a[...] + d_b[...]
    h = (m + h_in[...]) / (d + 1.0)
    if with_w2:
        h = lax.dot_general(h, W[...], (((1,), (1,)), ((), ())),
                            preferred_element_type=jnp.float32)
    t = h + b[...]
    mu = jnp.mean(t, axis=-1, keepdims=True)
    var = jnp.mean((t - mu) ** 2, axis=-1, keepdims=True)
    t = (t - mu) * lax.rsqrt(var + 1e-5) * g[...] + be[...]
    t = jnp.where(t > 0, t, jnp.exp(jnp.minimum(t, 0.0)) - 1.0)
    if with_w2:
        t = lax.dot_general(t, W2[...], (((1,), (1,)), ((), ())),
                            preferred_element_type=jnp.float32)
    out_ref[...] = t


def _make_tc_layer(n, d, with_w2, block_rows=1000):
    grid = n // block_rows
    row_spec = pl.BlockSpec((block_rows, d), lambda i: (i, 0))
    deg_spec = pl.BlockSpec((block_rows, 1), lambda i: (i, 0))
    full_spec = pl.BlockSpec((d, d), lambda i: (0, 0))
    vec_spec = pl.BlockSpec((1, d), lambda i: (0, 0))
    return pl.pallas_call(
        functools.partial(_layer_body, with_w2),
        grid=grid,
        in_specs=[row_spec, row_spec, row_spec, deg_spec, deg_spec,
                  full_spec, vec_spec, vec_spec, vec_spec, full_spec],
        out_specs=row_spec,
        out_shape=jax.ShapeDtypeStruct((n, d), jnp.float32),
    )


def kernel(features, edge_index, W1, b1, g1, be1, W2, b2, g2, be2):
    n, d = features.shape
    e = edge_index.shape[1]

    chunk = SB * NC * NS * 8
    e_pad = ((e + chunk - 1) // chunk) * chunk
    n_pad = ((n + 1 + NS * 8 - 1) // (NS * 8)) * (NS * 8)

    src = edge_index[0]
    dst = edge_index[1]
    pad = e_pad - e
    src2d = jnp.concatenate([src, jnp.zeros((pad,), jnp.int32)]).reshape(-1, SB)
    dst2d = jnp.concatenate([dst, jnp.full((pad,), n, jnp.int32)]).reshape(-1, SB)
    zeros = jnp.zeros((n_pad // NS, d), jnp.float32)
    ones = jnp.ones((SB, d), jnp.float32)

    segsum_deg = _make_sc_segsum(n_pad, e_pad, d, with_deg=True)
    segsum = _make_sc_segsum(n_pad, e_pad, d, with_deg=False)
    layer1 = _make_tc_layer(n, d, with_w2=True)
    layer2 = _make_tc_layer(n, d, with_w2=False)

    m1, degp = segsum_deg(features, src2d, dst2d, zeros, ones)
    d_a = degp[0:n, 0:1]
    d_b = degp[n_pad:n_pad + n, 0:1]
    W1r = W1.reshape(d, d)
    y2 = layer1(m1[0:n], m1[n_pad:n_pad + n], features, d_a, d_b,
                W1r, b1.reshape(1, d), g1.reshape(1, d), be1.reshape(1, d), W2)

    (m2,) = segsum(y2, src2d, dst2d, zeros, ones)
    out = layer2(m2[0:n], m2[n_pad:n_pad + n], y2, d_a, d_b,
                 W2, b2.reshape(1, d), g2.reshape(1, d), be2.reshape(1, d), W2)
    return out
